# Initial kernel scaffold; baseline (speedup 1.0000x reference)
import jax
import jax.numpy as jnp
from jax.experimental import pallas as pl


def kernel(x, edge_index, edge_attr, W1, W2, We, gamma, beta):
    raise NotImplementedError("write your pallas kernel here")


# speedup vs baseline: 4.7572x; 4.7572x over previous
import functools

import jax
import jax.numpy as jnp
from jax import lax
from jax.experimental import pallas as pl
from jax.experimental.pallas import tpu as pltpu
from jax.experimental.pallas import tpu_sc as plsc

N_NODES = 10000
N_EDGES = 320000
D_FEAT = 128
D_HALF = D_FEAT // 2
D_EDGE = 16
EPS = 1e-5

NC = 2
NS = 16
CHUNK = 400
EPT_X = N_EDGES // NS
NCH_X = EPT_X // CHUNK
EPT_E = N_EDGES // (NC * NS)
NCH_E = EPT_E // CHUNK
RPT = 624
TAIL = N_NODES - NS * RPT
TAIL0 = NS * RPT


def _sc_segment_sums(x2, src_arr, dst_arr, edge_attr, z64, z16):
    mesh = plsc.VectorSubcoreMesh(core_axis_name="c", subcore_axis_name="s")

    @functools.partial(
        pl.kernel,
        mesh=mesh,
        compiler_params=pltpu.CompilerParams(use_tc_tiling_on_sc=False),
        out_type=(
            jax.ShapeDtypeStruct((NC, N_NODES, D_HALF), jnp.float32),
            jax.ShapeDtypeStruct((NC, N_NODES, D_EDGE), jnp.float32),
        ),
        scratch_types=[
            pltpu.VMEM_SHARED((N_NODES, D_HALF), jnp.float32),
            pltpu.VMEM_SHARED((N_NODES, D_EDGE), jnp.float32),
            pltpu.VMEM((CHUNK,), jnp.int32),
            pltpu.VMEM((CHUNK,), jnp.int32),
            pltpu.VMEM((CHUNK,), jnp.int32),
            pltpu.VMEM((CHUNK, D_HALF), jnp.float32),
            pltpu.VMEM((CHUNK, D_EDGE), jnp.float32),
            pltpu.SemaphoreType.DMA,
        ],
    )
    def seg(x2_hbm, src_hbm, dst_hbm, ea_hbm, z64_hbm, z16_hbm, outx, oute,
            acc, eacc, src_v, dst_v, edst_v, rows_v, erows_v, sem):
        c = lax.axis_index("c")
        s = lax.axis_index("s")
        r0 = s * RPT

        pltpu.sync_copy(z64_hbm.at[pl.ds(r0, RPT)], acc.at[pl.ds(r0, RPT)])
        pltpu.sync_copy(z16_hbm.at[pl.ds(r0, RPT)], eacc.at[pl.ds(r0, RPT)])

        @pl.when(s == NS - 1)
        def _():
            pltpu.sync_copy(z64_hbm.at[pl.ds(TAIL0, TAIL)],
                            acc.at[pl.ds(TAIL0, TAIL)])
            pltpu.sync_copy(z16_hbm.at[pl.ds(TAIL0, TAIL)],
                            eacc.at[pl.ds(TAIL0, TAIL)])

        plsc.subcore_barrier()

        def body(j, carry):
            base = s * EPT_X + j * CHUNK
            pltpu.sync_copy(src_hbm.at[pl.ds(base, CHUNK)], src_v)
            pltpu.sync_copy(dst_hbm.at[pl.ds(base, CHUNK)], dst_v)
            pltpu.async_copy(x2_hbm.at[c].at[src_v], rows_v, sem).wait()
            pltpu.sync_copy(rows_v, acc.at[dst_v], add=True)

            @pl.when(j < NCH_E)
            def _():
                ebase = c * (N_EDGES // NC) + s * EPT_E + j * CHUNK
                pltpu.sync_copy(dst_hbm.at[pl.ds(ebase, CHUNK)], edst_v)
                pltpu.sync_copy(ea_hbm.at[pl.ds(ebase, CHUNK)], erows_v)
                pltpu.sync_copy(erows_v, eacc.at[edst_v], add=True)

            return carry

        lax.fori_loop(0, NCH_X, body, 0)
        plsc.subcore_barrier()

        pltpu.sync_copy(acc.at[pl.ds(r0, RPT)], outx.at[c, pl.ds(r0, RPT)])
        pltpu.sync_copy(eacc.at[pl.ds(r0, RPT)], oute.at[c, pl.ds(r0, RPT)])

        @pl.when(s == NS - 1)
        def _():
            pltpu.sync_copy(acc.at[pl.ds(TAIL0, TAIL)],
                            outx.at[c, pl.ds(TAIL0, TAIL)])
            pltpu.sync_copy(eacc.at[pl.ds(TAIL0, TAIL)],
                            oute.at[c, pl.ds(TAIL0, TAIL)])

    return seg(x2, src_arr, dst_arr, edge_attr, z64, z16)


def _tc_dense_body(px_ref, pe_ref, x_ref, w1_ref, w2_ref, we_ref,
                   gamma_ref, beta_ref, o_ref):
    eagg = pe_ref[0] + pe_ref[1]
    h = jnp.dot(px_ref[0], w1_ref[pl.ds(0, D_HALF), :],
                preferred_element_type=jnp.float32)
    h = h + jnp.dot(px_ref[1], w1_ref[pl.ds(D_HALF, D_HALF), :],
                    preferred_element_type=jnp.float32)
    h = h + jnp.dot(eagg, we_ref[...], preferred_element_type=jnp.float32)
    h = h + jnp.dot(x_ref[...], w2_ref[...], preferred_element_type=jnp.float32)
    mean = jnp.mean(h, axis=0, keepdims=True)
    d = h - mean
    var = jnp.mean(d * d, axis=0, keepdims=True)
    o = d * lax.rsqrt(var + EPS) * gamma_ref[...] + beta_ref[...]
    o_ref[...] = jnp.maximum(o, 0.0)


def _tc_dense(px, pe, x, W1, W2, We, gamma, beta):
    return pl.pallas_call(
        _tc_dense_body,
        out_shape=jax.ShapeDtypeStruct((N_NODES, D_FEAT), jnp.float32),
    )(px, pe, x, W1, W2, We, gamma, beta)


@jax.jit
def kernel(x, edge_index, edge_attr, W1, W2, We, gamma, beta):
    x2 = jnp.stack([x[:, :D_HALF], x[:, D_HALF:]])
    z64 = jnp.zeros((N_NODES, D_HALF), jnp.float32)
    z16 = jnp.zeros((N_NODES, D_EDGE), jnp.float32)
    px, pe = _sc_segment_sums(x2, edge_index[0], edge_index[1],
                              edge_attr, z64, z16)
    return _tc_dense(px, pe, x, W1, W2, We,
                     gamma.reshape(1, D_FEAT), beta.reshape(1, D_FEAT))


# speedup vs baseline: 7.0476x; 1.4815x over previous
import functools

import jax
import jax.numpy as jnp
from jax import lax
from jax.experimental import pallas as pl
from jax.experimental.pallas import tpu as pltpu
from jax.experimental.pallas import tpu_sc as plsc

N_NODES = 10000
N_EDGES = 320000
D_FEAT = 128
D_HALF = D_FEAT // 2
D_EDGE = 16
EPS = 1e-5

NC = 2
NS = 16
CHUNK = 400
EPT = N_EDGES // NS
NCH = EPT // CHUNK
RPT = 624
TAIL = N_NODES - NS * RPT
TAIL0 = NS * RPT


def _sc_segment_sums(xv, src_arr, dst_arr, edge_attr, z64, z16):
    mesh = plsc.VectorSubcoreMesh(core_axis_name="c", subcore_axis_name="s")

    @functools.partial(
        pl.kernel,
        mesh=mesh,
        compiler_params=pltpu.CompilerParams(use_tc_tiling_on_sc=False),
        out_type=(
            jax.ShapeDtypeStruct((NC, N_NODES, D_HALF), jnp.float32),
            jax.ShapeDtypeStruct((NC, N_NODES, D_EDGE), jnp.float32),
        ),
        scratch_types=[
            pltpu.VMEM_SHARED((N_NODES, D_HALF), jnp.float32),
            pltpu.VMEM_SHARED((N_NODES, D_EDGE), jnp.float32),
            pltpu.VMEM((CHUNK,), jnp.int32),
            pltpu.VMEM((CHUNK,), jnp.int32),
            pltpu.VMEM((CHUNK,), jnp.int32),
            pltpu.VMEM((CHUNK,), jnp.int32),
            pltpu.VMEM((CHUNK, D_HALF), jnp.float32),
            pltpu.VMEM((CHUNK, D_HALF), jnp.float32),
            pltpu.VMEM((CHUNK, D_EDGE), jnp.float32),
            pltpu.VMEM((CHUNK, D_EDGE), jnp.float32),
            pltpu.SemaphoreType.DMA,
            pltpu.SemaphoreType.DMA,
            pltpu.SemaphoreType.DMA,
            pltpu.SemaphoreType.DMA,
            pltpu.SemaphoreType.DMA,
            pltpu.SemaphoreType.DMA,
        ],
    )
    def seg(xv_hbm, src_hbm, dst_hbm, ea_hbm, z64_hbm, z16_hbm, outx, oute,
            acc, eacc, sidx0, sidx1, didx0, didx1, rows0, rows1,
            erows0, erows1, sg0, sg1, si0, si1, se0, se1):
        c = lax.axis_index("c")
        s = lax.axis_index("s")
        r0 = s * RPT
        base0 = s * EPT

        sidx = (sidx0, sidx1)
        didx = (didx0, didx1)
        rows = (rows0, rows1)
        erows = (erows0, erows1)
        sem_g = (sg0, sg1)
        sem_i = (si0, si1)
        sem_e = (se0, se1)

        def idx_copy(j, p):
            a = pltpu.make_async_copy(
                src_hbm.at[pl.ds(base0 + j * CHUNK, CHUNK)], sidx[p], sem_i[p])
            b = pltpu.make_async_copy(
                dst_hbm.at[pl.ds(base0 + j * CHUNK, CHUNK)], didx[p], sem_i[p])
            return a, b

        def ea_copy(j, p):
            return pltpu.make_async_copy(
                ea_hbm.at[pl.ds(base0 + j * CHUNK, CHUNK)], erows[p], sem_e[p])

        def gather(p):
            return pltpu.make_async_copy(xv_hbm.at[sidx[p]], rows[p], sem_g[p])

        def transform(p):
            for i in range(CHUNK // 16):
                sl = pl.ds(i * 16, 16)
                sidx[p][sl] = sidx[p][sl] * 2 + c

        a, b = idx_copy(0, 0)
        a.start(); b.start()
        a, b = idx_copy(1, 1)
        a.start(); b.start()

        @pl.when(c == 0)
        def _():
            ea_copy(0, 0).start()

        @pl.when(c == 1)
        def _():
            ea_copy(1, 1).start()

        pltpu.sync_copy(z64_hbm.at[pl.ds(r0, RPT)], acc.at[pl.ds(r0, RPT)])
        pltpu.sync_copy(z16_hbm.at[pl.ds(r0, RPT)], eacc.at[pl.ds(r0, RPT)])

        @pl.when(s == NS - 1)
        def _():
            pltpu.sync_copy(z64_hbm.at[pl.ds(TAIL0, TAIL)],
                            acc.at[pl.ds(TAIL0, TAIL)])
            pltpu.sync_copy(z16_hbm.at[pl.ds(TAIL0, TAIL)],
                            eacc.at[pl.ds(TAIL0, TAIL)])

        plsc.subcore_barrier()

        a, b = idx_copy(0, 0)
        a.wait(); b.wait()
        transform(0)
        gather(0).start()

        def emit_iter(j, p):
            q = 1 - p
            gather(p).wait()

            @pl.when(j + 1 < NCH)
            def _():
                a, b = idx_copy(j + 1, q)
                a.wait(); b.wait()
                transform(q)
                gather(q).start()

            pltpu.sync_copy(rows[p], acc.at[didx[p]], add=True)

            @pl.when(c == p)
            def _():
                ea_copy(j, p).wait()
                pltpu.sync_copy(erows[p], eacc.at[didx[p]], add=True)

            @pl.when(j + 2 < NCH)
            def _():
                a, b = idx_copy(j + 2, p)
                a.start(); b.start()

            @pl.when((c == p) & (j + 2 < NCH))
            def _():
                ea_copy(j + 2, p).start()

        def body(k, carry):
            emit_iter(2 * k, 0)
            emit_iter(2 * k + 1, 1)
            return carry

        lax.fori_loop(0, NCH // 2, body, 0)
        plsc.subcore_barrier()

        pltpu.sync_copy(acc.at[pl.ds(r0, RPT)], outx.at[c, pl.ds(r0, RPT)])
        pltpu.sync_copy(eacc.at[pl.ds(r0, RPT)], oute.at[c, pl.ds(r0, RPT)])

        @pl.when(s == NS - 1)
        def _():
            pltpu.sync_copy(acc.at[pl.ds(TAIL0, TAIL)],
                            outx.at[c, pl.ds(TAIL0, TAIL)])
            pltpu.sync_copy(eacc.at[pl.ds(TAIL0, TAIL)],
                            oute.at[c, pl.ds(TAIL0, TAIL)])

    return seg(xv, src_arr, dst_arr, edge_attr, z64, z16)


def _tc_dense_body(px_ref, pe_ref, x_ref, w1_ref, w2_ref, we_ref,
                   gamma_ref, beta_ref, o_ref):
    eagg = pe_ref[0] + pe_ref[1]
    h = jnp.dot(px_ref[0], w1_ref[pl.ds(0, D_HALF), :],
                preferred_element_type=jnp.float32)
    h = h + jnp.dot(px_ref[1], w1_ref[pl.ds(D_HALF, D_HALF), :],
                    preferred_element_type=jnp.float32)
    h = h + jnp.dot(eagg, we_ref[...], preferred_element_type=jnp.float32)
    h = h + jnp.dot(x_ref[...], w2_ref[...], preferred_element_type=jnp.float32)
    mean = jnp.mean(h, axis=0, keepdims=True)
    d = h - mean
    var = jnp.mean(d * d, axis=0, keepdims=True)
    o = d * lax.rsqrt(var + EPS) * gamma_ref[...] + beta_ref[...]
    o_ref[...] = jnp.maximum(o, 0.0)


def _tc_dense(px, pe, x, W1, W2, We, gamma, beta):
    return pl.pallas_call(
        _tc_dense_body,
        out_shape=jax.ShapeDtypeStruct((N_NODES, D_FEAT), jnp.float32),
    )(px, pe, x, W1, W2, We, gamma, beta)


@jax.jit
def kernel(x, edge_index, edge_attr, W1, W2, We, gamma, beta):
    xv = x.reshape(2 * N_NODES, D_HALF)
    z64 = jnp.zeros((N_NODES, D_HALF), jnp.float32)
    z16 = jnp.zeros((N_NODES, D_EDGE), jnp.float32)
    px, pe = _sc_segment_sums(xv, edge_index[0], edge_index[1],
                              edge_attr, z64, z16)
    return _tc_dense(px, pe, x, W1, W2, We,
                     gamma.reshape(1, D_FEAT), beta.reshape(1, D_FEAT))
